# Initial kernel scaffold; baseline (speedup 1.0000x reference)
import jax
import jax.numpy as jnp
from jax.experimental import pallas as pl


def kernel(input, table, W):
    raise NotImplementedError("write your pallas kernel here")


# speedup vs baseline: 7.2163x; 7.2163x over previous
import functools

import jax
import jax.numpy as jnp
from jax import lax
from jax.experimental import pallas as pl
from jax.experimental.pallas import tpu as pltpu
from jax.experimental.pallas import tpu_sc as plsc

NUM_EMB = 100000
EMB_DIM = 64
BASE_DIM = 128
BATCH = 4096
HIST = 50

_INFO = plsc.get_sparse_core_info()
_NC, _NS = _INFO.num_cores, _INFO.num_subcores
_NW = _NC * _NS
_BAGS_PER_W = BATCH // _NW
_LANES = 16
_VPR = EMB_DIM // _LANES


def _sc_bagsum(table, idx):
    mesh = plsc.VectorSubcoreMesh(core_axis_name="c", subcore_axis_name="s")

    @functools.partial(
        pl.kernel,
        mesh=mesh,
        out_type=jax.ShapeDtypeStruct((BATCH, EMB_DIM), jnp.float32),
        compiler_params=pltpu.CompilerParams(use_tc_tiling_on_sc=False),
        scratch_types=[
            pltpu.VMEM((_BAGS_PER_W, HIST), jnp.int32),
            pltpu.VMEM((HIST, EMB_DIM), jnp.float32),
            pltpu.VMEM((HIST, EMB_DIM), jnp.float32),
            pltpu.VMEM((_BAGS_PER_W, EMB_DIM), jnp.float32),
            pltpu.SemaphoreType.DMA,
            pltpu.SemaphoreType.DMA,
        ],
    )
    def sc_kernel(table_hbm, idx_hbm, out_hbm, idx_v, rows0, rows1, out_v,
                  sem0, sem1):
        wid = lax.axis_index("s") * _NC + lax.axis_index("c")
        base = wid * _BAGS_PER_W
        pltpu.sync_copy(idx_hbm.at[pl.ds(base, _BAGS_PER_W)], idx_v)

        rows = (rows0, rows1)
        sems = (sem0, sem1)

        def gather(b, p):
            pltpu.async_copy(table_hbm.at[idx_v.at[b]], rows[p], sems[p])

        def drain(p):
            pltpu.make_async_copy(table_hbm.at[idx_v.at[0]], rows[p],
                                  sems[p]).wait()

        def accumulate(b, p):
            rbuf = rows[p]

            def hbody(h, acc):
                return tuple(
                    acc[i] + rbuf[h, pl.ds(i * _LANES, _LANES)]
                    for i in range(_VPR)
                )

            acc = lax.fori_loop(
                0, HIST, hbody,
                tuple(jnp.zeros((_LANES,), jnp.float32) for _ in range(_VPR)))
            for i in range(_VPR):
                out_v[b, pl.ds(i * _LANES, _LANES)] = acc[i]

        gather(0, 0)
        gather(1, 1)

        def body(it, _):
            b0 = it * 2
            drain(0)
            accumulate(b0, 0)

            @pl.when(b0 + 2 < _BAGS_PER_W)
            def _():
                gather(b0 + 2, 0)

            drain(1)
            accumulate(b0 + 1, 1)

            @pl.when(b0 + 3 < _BAGS_PER_W)
            def _():
                gather(b0 + 3, 1)

            return 0

        lax.fori_loop(0, _BAGS_PER_W // 2, body, 0)
        pltpu.sync_copy(out_v, out_hbm.at[pl.ds(base, _BAGS_PER_W)])

    return sc_kernel


def _proj_body(x_ref, w_ref, o_ref):
    o_ref[...] = lax.dot_general(
        x_ref[...], w_ref[...],
        (((1,), (1,)), ((), ())),
        preferred_element_type=jnp.float32,
        precision=lax.Precision.HIGHEST,
    )


def _tc_proj(emb, W):
    blk = 1024
    return pl.pallas_call(
        _proj_body,
        grid=(BATCH // blk,),
        in_specs=[
            pl.BlockSpec((blk, EMB_DIM), lambda i: (i, 0)),
            pl.BlockSpec((BASE_DIM, EMB_DIM), lambda i: (0, 0)),
        ],
        out_specs=pl.BlockSpec((blk, BASE_DIM), lambda i: (i, 0)),
        out_shape=jax.ShapeDtypeStruct((BATCH, BASE_DIM), jnp.float32),
    )(emb, W)


def kernel(input, table, W):
    idx = input.astype(jnp.int32)
    emb = _sc_bagsum(table, idx)(table, idx)
    return _tc_proj(emb, W)


# speedup vs baseline: 8.6708x; 1.2015x over previous
import functools

import jax
import jax.numpy as jnp
from jax import lax
from jax.experimental import pallas as pl
from jax.experimental.pallas import tpu as pltpu
from jax.experimental.pallas import tpu_sc as plsc

NUM_EMB = 100000
EMB_DIM = 64
BASE_DIM = 128
BATCH = 4096
HIST = 50

_INFO = plsc.get_sparse_core_info()
_NC, _NS = _INFO.num_cores, _INFO.num_subcores
_NW = _NC * _NS
_BAGS_PER_W = BATCH // _NW
_LANES = 16
_VPR = EMB_DIM // _LANES


def _sc_bagsum(table, idx):
    mesh = plsc.VectorSubcoreMesh(core_axis_name="c", subcore_axis_name="s")

    @functools.partial(
        pl.kernel,
        mesh=mesh,
        out_type=jax.ShapeDtypeStruct((BATCH, EMB_DIM), jnp.float32),
        compiler_params=pltpu.CompilerParams(use_tc_tiling_on_sc=False),
        scratch_types=[
            pltpu.VMEM((_BAGS_PER_W, HIST), jnp.int32),
            pltpu.VMEM((HIST, EMB_DIM), jnp.float32),
            pltpu.VMEM((HIST, EMB_DIM), jnp.float32),
            pltpu.VMEM((HIST, EMB_DIM), jnp.float32),
            pltpu.VMEM((HIST, EMB_DIM), jnp.float32),
            pltpu.VMEM((_BAGS_PER_W, EMB_DIM), jnp.float32),
            pltpu.SemaphoreType.DMA,
            pltpu.SemaphoreType.DMA,
            pltpu.SemaphoreType.DMA,
            pltpu.SemaphoreType.DMA,
        ],
    )
    def sc_kernel(table_hbm, idx_hbm, out_hbm, idx_v, rows0, rows1, rows2,
                  rows3, out_v, sem0, sem1, sem2, sem3):
        wid = lax.axis_index("s") * _NC + lax.axis_index("c")
        base = wid * _BAGS_PER_W
        pltpu.sync_copy(idx_hbm.at[pl.ds(base, _BAGS_PER_W)], idx_v)

        rows = (rows0, rows1, rows2, rows3)
        sems = (sem0, sem1, sem2, sem3)
        nbuf = 4

        def gather(b, p):
            pltpu.async_copy(table_hbm.at[idx_v.at[b]], rows[p], sems[p])

        def drain(p):
            pltpu.make_async_copy(table_hbm.at[idx_v.at[0]], rows[p],
                                  sems[p]).wait()

        def accumulate(b, p):
            rbuf = rows[p]
            acc0 = [rbuf[0, pl.ds(i * _LANES, _LANES)] for i in range(_VPR)]
            acc1 = [rbuf[1, pl.ds(i * _LANES, _LANES)] for i in range(_VPR)]
            for h in range(2, HIST, 2):
                for i in range(_VPR):
                    acc0[i] = acc0[i] + rbuf[h, pl.ds(i * _LANES, _LANES)]
                for i in range(_VPR):
                    acc1[i] = acc1[i] + rbuf[h + 1, pl.ds(i * _LANES, _LANES)]
            for i in range(_VPR):
                out_v[b, pl.ds(i * _LANES, _LANES)] = acc0[i] + acc1[i]

        for p in range(nbuf):
            gather(p, p)

        def body(it, _):
            b0 = it * nbuf
            for p in range(nbuf):
                drain(p)
                accumulate(b0 + p, p)

                @pl.when(b0 + p + nbuf < _BAGS_PER_W)
                def _():
                    gather(b0 + p + nbuf, p)

            return 0

        lax.fori_loop(0, _BAGS_PER_W // nbuf, body, 0)
        pltpu.sync_copy(out_v, out_hbm.at[pl.ds(base, _BAGS_PER_W)])

    return sc_kernel


def _proj_body(x_ref, w_ref, o_ref):
    o_ref[...] = lax.dot_general(
        x_ref[...], w_ref[...],
        (((1,), (1,)), ((), ())),
        preferred_element_type=jnp.float32,
        precision=lax.Precision.HIGHEST,
    )


def _tc_proj(emb, W):
    blk = 1024
    return pl.pallas_call(
        _proj_body,
        grid=(BATCH // blk,),
        in_specs=[
            pl.BlockSpec((blk, EMB_DIM), lambda i: (i, 0)),
            pl.BlockSpec((BASE_DIM, EMB_DIM), lambda i: (0, 0)),
        ],
        out_specs=pl.BlockSpec((blk, BASE_DIM), lambda i: (i, 0)),
        out_shape=jax.ShapeDtypeStruct((BATCH, BASE_DIM), jnp.float32),
    )(emb, W)


def kernel(input, table, W):
    idx = input.astype(jnp.int32)
    emb = _sc_bagsum(table, idx)(table, idx)
    return _tc_proj(emb, W)


# speedup vs baseline: 9.0351x; 1.0420x over previous
import functools

import jax
import jax.numpy as jnp
from jax import lax
from jax.experimental import pallas as pl
from jax.experimental.pallas import tpu as pltpu
from jax.experimental.pallas import tpu_sc as plsc

NUM_EMB = 100000
EMB_DIM = 64
BASE_DIM = 128
BATCH = 4096
HIST = 50

_INFO = plsc.get_sparse_core_info()
_NC, _NS = _INFO.num_cores, _INFO.num_subcores
_NW = _NC * _NS
_BAGS_PER_W = BATCH // _NW
_IDX_PER_W = _BAGS_PER_W * HIST
_CHUNK_BAGS = 8
_CHUNK_ROWS = _CHUNK_BAGS * HIST
_LANES = 16
_VPR = EMB_DIM // _LANES


def _sc_bagsum(table, idx):
    mesh = plsc.VectorSubcoreMesh(core_axis_name="c", subcore_axis_name="s")

    @functools.partial(
        pl.kernel,
        mesh=mesh,
        out_type=jax.ShapeDtypeStruct((BATCH, EMB_DIM), jnp.float32),
        compiler_params=pltpu.CompilerParams(use_tc_tiling_on_sc=False),
        scratch_types=[
            pltpu.VMEM((_IDX_PER_W,), jnp.int32),
            pltpu.VMEM((_CHUNK_ROWS, EMB_DIM), jnp.float32),
            pltpu.VMEM((_CHUNK_ROWS, EMB_DIM), jnp.float32),
            pltpu.VMEM((_BAGS_PER_W, EMB_DIM), jnp.float32),
            pltpu.SemaphoreType.DMA,
            pltpu.SemaphoreType.DMA,
        ],
    )
    def sc_kernel(table_hbm, idx_hbm, out_hbm, idx_v, rows0, rows1, out_v,
                  sem0, sem1):
        wid = lax.axis_index("s") * _NC + lax.axis_index("c")
        base = wid * _BAGS_PER_W
        pltpu.sync_copy(idx_hbm.at[pl.ds(wid * _IDX_PER_W, _IDX_PER_W)], idx_v)

        rows = (rows0, rows1)
        sems = (sem0, sem1)

        def gather(c, p):
            off = c * _CHUNK_ROWS
            for g0 in range(0, _CHUNK_ROWS, 128):
                glen = min(128, _CHUNK_ROWS - g0)
                pltpu.async_copy(
                    table_hbm.at[idx_v.at[pl.ds(off + g0, glen)]],
                    rows[p].at[pl.ds(g0, glen)],
                    sems[p])

        def drain(p):
            for g0 in range(0, _CHUNK_ROWS, 128):
                glen = min(128, _CHUNK_ROWS - g0)
                pltpu.make_async_copy(
                    table_hbm.at[idx_v.at[pl.ds(g0, glen)]],
                    rows[p].at[pl.ds(g0, glen)],
                    sems[p]).wait()

        def accumulate(c, p):
            rbuf = rows[p]

            def bag_body(j, _):
                r0 = j * HIST
                acc0 = [rbuf[r0, pl.ds(i * _LANES, _LANES)]
                        for i in range(_VPR)]
                acc1 = [rbuf[r0 + 1, pl.ds(i * _LANES, _LANES)]
                        for i in range(_VPR)]
                for h in range(2, HIST, 2):
                    for i in range(_VPR):
                        acc0[i] = acc0[i] + rbuf[r0 + h,
                                                 pl.ds(i * _LANES, _LANES)]
                    for i in range(_VPR):
                        acc1[i] = acc1[i] + rbuf[r0 + h + 1,
                                                 pl.ds(i * _LANES, _LANES)]
                b = c * _CHUNK_BAGS + j
                for i in range(_VPR):
                    out_v[b, pl.ds(i * _LANES, _LANES)] = acc0[i] + acc1[i]
                return 0

            lax.fori_loop(0, _CHUNK_BAGS, bag_body, 0)

        nchunks = _BAGS_PER_W // _CHUNK_BAGS
        gather(0, 0)
        gather(1, 1)

        def body(c, _):
            p = 0
            drain(p)
            accumulate(c, p)

            @pl.when(c + 2 < nchunks)
            def _():
                gather(c + 2, p)

            p = 1
            drain(p)
            accumulate(c + 1, p)

            @pl.when(c + 3 < nchunks)
            def _():
                gather(c + 3, p)

            return 0

        lax.fori_loop(0, nchunks // 2, lambda it, x: body(it * 2, x), 0)
        pltpu.sync_copy(out_v, out_hbm.at[pl.ds(base, _BAGS_PER_W)])

    return sc_kernel


def _proj_body(x_ref, w_ref, o_ref):
    o_ref[...] = lax.dot_general(
        x_ref[...], w_ref[...],
        (((1,), (1,)), ((), ())),
        preferred_element_type=jnp.float32,
        precision=lax.Precision.HIGHEST,
    )


def _tc_proj(emb, W):
    blk = 1024
    return pl.pallas_call(
        _proj_body,
        grid=(BATCH // blk,),
        in_specs=[
            pl.BlockSpec((blk, EMB_DIM), lambda i: (i, 0)),
            pl.BlockSpec((BASE_DIM, EMB_DIM), lambda i: (0, 0)),
        ],
        out_specs=pl.BlockSpec((blk, BASE_DIM), lambda i: (i, 0)),
        out_shape=jax.ShapeDtypeStruct((BATCH, BASE_DIM), jnp.float32),
    )(emb, W)


def kernel(input, table, W):
    idx = input.astype(jnp.int32).reshape(-1)
    emb = _sc_bagsum(table, idx)(table, idx)
    return _tc_proj(emb, W)
